# baseline (speedup 1.0000x reference)
import functools

import jax
import jax.numpy as jnp
from jax import lax
from jax.experimental import pallas as pl
from jax.experimental.pallas import tpu as pltpu
from jax.experimental.pallas import tpu_sc as plsc

_N, _D, _E = 10000, 128, 320000
_NC, _NS = 2, 16
_HROW = 5120
_PACKW = 128
_CHUNK = 64
_EPAD = 327680
_NSLAB = _EPAD // _PACKW
_SLABMAX = 176
_SROWS = 5248
_RPS = _SROWS // _NS
_PADROW = 10300


def _gather_add_body(x_hbm, pack_hbm, bounds_hbm, zeros_hbm, out_hbm,
                     bbuf, packv, colb0, colb1, rowb0, rowb1, gbuf, acc,
                     sem0, sem1):
    cid = lax.axis_index("c")
    sid = lax.axis_index("s")

    pltpu.sync_copy(bounds_hbm.at[pl.ds((cid * _NS + sid) * 16, 16)], bbuf)
    bv = bbuf[...]
    lane = lax.iota(jnp.int32, 16)
    neg = jnp.full((16,), -0x80000000, jnp.int32)
    lo = jnp.max(jnp.where(lane == 0, bv, neg))
    hi = jnp.max(jnp.where(lane == 1, bv, neg))
    ns = hi - lo
    lo = pl.multiple_of(lo, 8)
    pltpu.sync_copy(pack_hbm.at[pl.ds(lo, _SLABMAX)], packv)
    pltpu.sync_copy(zeros_hbm, acc.at[pl.ds(sid * _RPS, _RPS)])
    plsc.subcore_barrier()

    sems = (sem0, sem1)
    colbs = (colb0, colb1)
    rowbs = (rowb0, rowb1)
    base = cid * _HROW

    def unpack(j, q):
        for k in range(_CHUNK // 16):
            v = packv[j, pl.ds(q * _CHUNK + k * 16, 16)]
            rl = (v >> 16) - base
            valid = (rl >= 0) & (rl < _HROW)
            rowbs[q][pl.ds(k * 16, 16)] = jnp.where(valid, rl, _HROW)
            colbs[q][pl.ds(k * 16, 16)] = v & 0xFFFF

    @pl.when(ns > 0)
    def _():
        for q in (0, 1):
            unpack(0, q)
            pltpu.async_copy(x_hbm.at[colbs[q]], gbuf.at[q], sems[q])

    def step(g, carry):
        for q in (0, 1):
            pltpu.make_async_copy(x_hbm.at[colbs[q]], gbuf.at[q],
                                  sems[q]).wait()
            pltpu.sync_copy(gbuf.at[q], acc.at[rowbs[q]], add=True)

            @pl.when(g + 1 < ns)
            def _():
                unpack(g + 1, q)
                pltpu.async_copy(x_hbm.at[colbs[q]], gbuf.at[q], sems[q])
        return carry

    lax.fori_loop(0, ns, step, 0)
    plsc.subcore_barrier()

    pltpu.sync_copy(acc.at[pl.ds(sid * _RPS, _RPS)],
                    out_hbm.at[pl.ds(cid * _SROWS + sid * _RPS, _RPS)])


_ga_kernel = functools.partial(
    pl.kernel,
    out_type=jax.ShapeDtypeStruct((_NC * _SROWS, _D), jnp.float32),
    mesh=plsc.VectorSubcoreMesh(core_axis_name="c", subcore_axis_name="s",
                                num_cores=_NC, num_subcores=_NS),
    compiler_params=pltpu.CompilerParams(needs_layout_passes=False),
    scratch_types=[
        pltpu.VMEM((16,), jnp.int32),
        pltpu.VMEM((_SLABMAX, _PACKW), jnp.int32),
        pltpu.VMEM((_CHUNK,), jnp.int32),
        pltpu.VMEM((_CHUNK,), jnp.int32),
        pltpu.VMEM((_CHUNK,), jnp.int32),
        pltpu.VMEM((_CHUNK,), jnp.int32),
        pltpu.VMEM((2, _CHUNK, _D), jnp.float32),
        pltpu.VMEM_SHARED((_SROWS, _D), jnp.float32),
        pltpu.SemaphoreType.DMA,
        pltpu.SemaphoreType.DMA,
    ],
)(_gather_add_body)


def kernel(r0, edge_index, train_mask):
    row = edge_index[0]
    col = edge_index[1]
    pad = _EPAD - _E
    rowp = jnp.concatenate([row, jnp.full((pad,), _PADROW, jnp.int32)])
    colp = jnp.concatenate([col, jnp.zeros((pad,), jnp.int32)])
    packed = jnp.sort((rowp << 16) | colp)
    tail = jnp.full((_SLABMAX * _PACKW,), (_PADROW << 16), jnp.int32)
    packp = jnp.concatenate([packed, tail]).reshape(_NSLAB + _SLABMAX, _PACKW)

    cnt = jnp.searchsorted(packed, jnp.int32(_HROW << 16)).astype(jnp.int32)
    k = jnp.arange(_NS + 1, dtype=jnp.int32)
    a_c = jnp.stack([jnp.zeros((), jnp.int32), (cnt // _PACKW) & ~7])
    b_c = jnp.stack([(cnt + _PACKW - 1) // _PACKW,
                     jnp.full((), _NSLAB, jnp.int32)])
    span = b_c - a_c
    ali = ((span[:, None] * k[None, :]) // _NS) & ~7
    ali = ali.at[:, _NS].set(span)
    cuts = a_c[:, None] + ali
    bounds = jnp.zeros((_NC, _NS, 16), jnp.int32)
    bounds = bounds.at[:, :, 0].set(cuts[:, :_NS])
    bounds = bounds.at[:, :, 1].set(cuts[:, 1:])
    bounds = bounds.reshape(_NC * _NS * 16)
    zeros = jnp.zeros((_RPS, _D), jnp.float32)

    def S(x):
        p = _ga_kernel(x, packp, bounds, zeros)
        return jnp.concatenate([p[:_HROW], p[_SROWS:_SROWS + _N - _HROW]],
                               axis=0)

    deg = S(jnp.ones((_N, _D), jnp.float32))[:, 0]
    dinv = jnp.where(deg > 0, 1.0 / jnp.sqrt(jnp.maximum(deg, 1e-12)), 0.0)
    maskf = train_mask.astype(jnp.float32)
    in_scale = (maskf * dinv)[:, None]
    mid_scale = (dinv * dinv)[:, None]
    out_scale = (-dinv)[:, None]

    def apply_L(r):
        z = S(in_scale * r)
        z = S(mid_scale * z)
        return out_scale * z

    dt = 0.2
    out = [r0]
    r = r0
    for _ in range(5):
        s1 = apply_L(r)
        s2 = apply_L(r + 0.5 * dt * s1)
        s3 = apply_L(r + 0.5 * dt * s2)
        s4 = apply_L(r + dt * s3)
        r = r + dt / 6.0 * (s1 + 2.0 * s2 + 2.0 * s3 + s4)
        out.append(r)
    return jnp.stack(out, axis=0)


# speedup vs baseline: 1.1955x; 1.1955x over previous
import functools

import jax
import jax.numpy as jnp
from jax import lax
from jax.experimental import pallas as pl
from jax.experimental.pallas import tpu as pltpu
from jax.experimental.pallas import tpu_sc as plsc

_N, _D, _E = 10000, 128, 320000
_NC, _NS = 2, 16
_HD = _D // _NC
_PACKW = 128
_CHUNK = 64
_NCHUNK_W = 160
_NBUF = 4
_NG = 2 * _NCHUNK_W // _NBUF
_EPAD = _NS * _NCHUNK_W * _PACKW
_NROWS = 10240
_RPS = _NROWS // _NS


def _gather_add_body(x_hbm, pack_hbm, zeros_hbm, out_hbm,
                     packv, colbM, rowbM, gbuf, acc, sem0, sem1, sem2, sem3):
    cid = lax.axis_index("c")
    sid = lax.axis_index("s")

    pltpu.sync_copy(pack_hbm.at[sid], packv)
    pltpu.sync_copy(zeros_hbm, acc.at[pl.ds(sid * _RPS, _RPS)])
    plsc.subcore_barrier()

    sems = (sem0, sem1, sem2, sem3)

    def unpack(j, h, q):
        for k in range(_CHUNK // 16):
            v = packv[j, pl.ds(h * _CHUNK + k * 16, 16)]
            colbM[q, pl.ds(k * 16, 16)] = ((v & 0xFFFF) << 1) | cid
            rowbM[q, pl.ds(k * 16, 16)] = v >> 16

    for q in range(_NBUF):
        unpack(q // 2, q % 2, q)
        pltpu.async_copy(x_hbm.at[colbM.at[q]], gbuf.at[q], sems[q])

    def step(g, carry):
        for q in range(_NBUF):
            pltpu.make_async_copy(x_hbm.at[colbM.at[q]], gbuf.at[q],
                                  sems[q]).wait()
            pltpu.sync_copy(gbuf.at[q], acc.at[rowbM.at[q]], add=True)

            @pl.when(g + 1 < _NG)
            def _():
                c = _NBUF * (g + 1) + q
                unpack(c // 2, q % 2, q)
                pltpu.async_copy(x_hbm.at[colbM.at[q]], gbuf.at[q], sems[q])
        return carry

    lax.fori_loop(0, _NG, step, 0)
    plsc.subcore_barrier()

    pltpu.sync_copy(acc.at[pl.ds(sid * _RPS, _RPS)],
                    out_hbm.at[pl.ds(cid * _NROWS + sid * _RPS, _RPS)])


_ga_kernel = functools.partial(
    pl.kernel,
    out_type=jax.ShapeDtypeStruct((_NC * _NROWS, _HD), jnp.float32),
    mesh=plsc.VectorSubcoreMesh(core_axis_name="c", subcore_axis_name="s",
                                num_cores=_NC, num_subcores=_NS),
    compiler_params=pltpu.CompilerParams(use_tc_tiling_on_sc=False),
    scratch_types=[
        pltpu.VMEM((_NCHUNK_W, _PACKW), jnp.int32),
        pltpu.VMEM((_NBUF, _CHUNK), jnp.int32),
        pltpu.VMEM((_NBUF, _CHUNK), jnp.int32),
        pltpu.VMEM((_NBUF, _CHUNK, _HD), jnp.float32),
        pltpu.VMEM_SHARED((_NROWS, _HD), jnp.float32),
        pltpu.SemaphoreType.DMA,
        pltpu.SemaphoreType.DMA,
        pltpu.SemaphoreType.DMA,
        pltpu.SemaphoreType.DMA,
    ],
)(_gather_add_body)


def kernel(r0, edge_index, train_mask):
    row = edge_index[0]
    col = edge_index[1]
    pad = _EPAD - _E
    rowp = jnp.concatenate([row, jnp.full((pad,), _N, jnp.int32)])
    colp = jnp.concatenate([col, jnp.zeros((pad,), jnp.int32)])
    packp = ((rowp << 16) | colp).reshape(_NS, _NCHUNK_W, _PACKW)
    zeros = jnp.zeros((_RPS, _HD), jnp.float32)

    def S(x):
        p = _ga_kernel(x.reshape(_NC * _N, _HD), packp, zeros)
        return jnp.concatenate([p[:_N], p[_NROWS:_NROWS + _N]], axis=1)

    deg = S(jnp.ones((_N, _D), jnp.float32))[:, 0]
    dinv = jnp.where(deg > 0, 1.0 / jnp.sqrt(jnp.maximum(deg, 1e-12)), 0.0)
    maskf = train_mask.astype(jnp.float32)
    in_scale = (maskf * dinv)[:, None]
    mid_scale = (dinv * dinv)[:, None]
    out_scale = (-dinv)[:, None]

    def apply_L(r):
        z = S(in_scale * r)
        z = S(mid_scale * z)
        return out_scale * z

    dt = 0.2
    out = [r0]
    r = r0
    for _ in range(5):
        s1 = apply_L(r)
        s2 = apply_L(r + 0.5 * dt * s1)
        s3 = apply_L(r + 0.5 * dt * s2)
        s4 = apply_L(r + dt * s3)
        r = r + dt / 6.0 * (s1 + 2.0 * s2 + 2.0 * s3 + s4)
        out.append(r)
    return jnp.stack(out, axis=0)
